# baseline (speedup 1.0000x reference)
import jax
import jax.numpy as jnp
from jax import lax
from jax.experimental import pallas as pl
from jax.experimental.pallas import tpu as pltpu
from jax.experimental.pallas import tpu_sc as plsc

_NC = 2
_NS = 16
_L = 16
_NW = _NC * _NS

_C1 = 36313
_C2 = 27191


def _sc_gather(x_flat, embed, seq_len):
    Tc = x_flat.shape[0]
    V, D = embed.shape
    mod = V - 1
    PW = Tc // _NW
    CH = min(PW, 64)
    NCHUNK = PW // CH
    assert PW % _L == 0 and PW * _NW == Tc and PW % CH == 0

    def body(x_hbm, emb_hbm, out_hbm, xbuf, hbuf, rows, sem, sem_o0, sem_o1):
        cid = lax.axis_index("c")
        sid = lax.axis_index("s")
        wid = sid * _NC + cid
        base = wid * PW

        pltpu.sync_copy(x_hbm.at[pl.ds(base, PW)], xbuf.at[pl.ds(_L, PW)])

        @pl.when(lax.rem(base, seq_len) != 0)
        def _():
            pltpu.sync_copy(x_hbm.at[pl.ds(base - _L, _L)], xbuf.at[pl.ds(0, _L)])

        pc = (1 << 17) % mod
        lanes = lax.iota(jnp.int32, _L)
        for i in range(PW // _L):
            cur = xbuf[pl.ds(_L + i * _L, _L)]
            prev = xbuf[pl.ds(_L - 1 + i * _L, _L)]
            r = (cur * _C1) ^ (prev * _C2)
            for _ in range(7):
                r = (r >> 17) * pc + (r & 0x1FFFF)
            r = jnp.where(r < 0, r + mod, r)
            r = jnp.where(r < 0, r + mod, r)
            r = jnp.where(r >= mod, r - mod, r)
            r = jnp.where(r >= mod, r - mod, r)
            pos = base + (i * _L) + lanes
            if seq_len & (seq_len - 1) == 0:
                pos_in_row = pos & (seq_len - 1)
            else:
                pos_in_row = lax.rem(pos, seq_len)
            hbuf[pl.ds(i * _L, _L)] = jnp.where(pos_in_row == 0, mod, r)

        out_sems = (sem_o0, sem_o1)
        drains = [None, None]
        for c in range(NCHUNK):
            b = c % 2
            if drains[b] is not None:
                drains[b].wait()
            idx = hbuf.at[pl.ds(c * CH, CH)]
            pltpu.async_copy(emb_hbm.at[idx], rows.at[b], sem).wait()
            drains[b] = pltpu.async_copy(
                rows.at[b], out_hbm.at[pl.ds(base + c * CH, CH)], out_sems[b])
        for b in range(2):
            if drains[b] is not None:
                drains[b].wait()

    run = pl.kernel(
        body,
        out_type=jax.ShapeDtypeStruct((Tc, D), jnp.float32),
        mesh=plsc.VectorSubcoreMesh(
            core_axis_name="c", subcore_axis_name="s",
            num_cores=_NC, num_subcores=_NS,
        ),
        scratch_types=[
            pltpu.VMEM((PW + _L,), jnp.int32),
            pltpu.VMEM((PW,), jnp.int32),
            pltpu.VMEM((2, CH, D), jnp.float32),
            pltpu.SemaphoreType.DMA,
            pltpu.SemaphoreType.DMA,
            pltpu.SemaphoreType.DMA,
        ],
    )
    return run(x_flat, embed)


def _tc_project_chunk(out_buf, e, wb, scale_arr, row_start, total_rows):
    S, D = e.shape
    MD = wb.shape[0]
    BM = 1024
    nblk = S // BM
    blk0 = row_start // BM
    assert row_start % BM == 0

    def body(*refs):
        s_ref, e_ref, w_ref, o_ref = refs[-4:]
        eb = (e_ref[...] * s_ref[0]).astype(jnp.bfloat16)
        o_ref[...] = lax.dot_general(
            eb, w_ref[...], (((1,), (1,)), ((), ())),
            preferred_element_type=jnp.float32,
        )

    data_specs = [
        pl.BlockSpec(memory_space=pltpu.SMEM),
        pl.BlockSpec((BM, D), lambda i: (i, 0)),
        pl.BlockSpec((MD, D), lambda i: (0, 0)),
    ]
    out_spec = pl.BlockSpec((BM, MD), lambda i, b0=blk0: (b0 + i, 0))
    if out_buf is None:
        return pl.pallas_call(
            body,
            grid=(nblk,),
            in_specs=data_specs,
            out_specs=out_spec,
            out_shape=jax.ShapeDtypeStruct((total_rows, MD), jnp.float32),
        )(scale_arr, e, wb)
    return pl.pallas_call(
        body,
        grid=(nblk,),
        in_specs=[pl.BlockSpec(memory_space=pl.ANY)] + data_specs,
        out_specs=out_spec,
        out_shape=jax.ShapeDtypeStruct((total_rows, MD), jnp.float32),
        input_output_aliases={0: 0},
    )(out_buf, scale_arr, e, wb)


def kernel(x, embed, proj_w, scale):
    B, S = x.shape
    MD = proj_w.shape[0]
    x_flat = x.reshape(-1).astype(jnp.int32)
    wb = proj_w.astype(jnp.bfloat16)
    scale_arr = jnp.asarray(scale, jnp.float32).reshape(1)
    T = B * S
    sizes = [S, T - S] if B > 1 else [T]
    starts = [0, S]
    es = [_sc_gather(lax.dynamic_slice_in_dim(x_flat, starts[c], sizes[c]), embed, S)
          for c in range(len(sizes))]
    out = None
    for c in range(len(sizes)):
        out = _tc_project_chunk(out, es[c], wb, scale_arr, starts[c], T)
    return out.reshape(B, S, MD)


# speedup vs baseline: 1.0366x; 1.0366x over previous
import jax
import jax.numpy as jnp
from jax import lax
from jax.experimental import pallas as pl
from jax.experimental.pallas import tpu as pltpu
from jax.experimental.pallas import tpu_sc as plsc

_NC = 2
_NS = 16
_L = 16
_NW = _NC * _NS

_C1 = 36313
_C2 = 27191


def _sc_gather(x_flat, embed, seq_len):
    Tc = x_flat.shape[0]
    V, D = embed.shape
    mod = V - 1
    PW = Tc // _NW
    CH = min(PW, 64)
    NCHUNK = PW // CH
    assert PW % _L == 0 and PW * _NW == Tc and PW % CH == 0

    def body(x_hbm, emb_hbm, out_hbm, xbuf, hbuf, rows, sem, sem_o0, sem_o1):
        cid = lax.axis_index("c")
        sid = lax.axis_index("s")
        wid = sid * _NC + cid
        base = wid * PW

        pltpu.sync_copy(x_hbm.at[pl.ds(base, PW)], xbuf.at[pl.ds(_L, PW)])

        @pl.when(lax.rem(base, seq_len) != 0)
        def _():
            pltpu.sync_copy(x_hbm.at[pl.ds(base - _L, _L)], xbuf.at[pl.ds(0, _L)])

        pc = (1 << 17) % mod
        lanes = lax.iota(jnp.int32, _L)
        for i in range(PW // _L):
            cur = xbuf[pl.ds(_L + i * _L, _L)]
            prev = xbuf[pl.ds(_L - 1 + i * _L, _L)]
            r = (cur * _C1) ^ (prev * _C2)
            for _ in range(7):
                r = (r >> 17) * pc + (r & 0x1FFFF)
            r = jnp.where(r < 0, r + mod, r)
            r = jnp.where(r < 0, r + mod, r)
            r = jnp.where(r >= mod, r - mod, r)
            r = jnp.where(r >= mod, r - mod, r)
            pos = base + (i * _L) + lanes
            if seq_len & (seq_len - 1) == 0:
                pos_in_row = pos & (seq_len - 1)
            else:
                pos_in_row = lax.rem(pos, seq_len)
            hbuf[pl.ds(i * _L, _L)] = jnp.where(pos_in_row == 0, mod, r)

        out_sems = (sem_o0, sem_o1)
        drains = [None, None]
        for c in range(NCHUNK):
            b = c % 2
            if drains[b] is not None:
                drains[b].wait()
            idx = hbuf.at[pl.ds(c * CH, CH)]
            pltpu.async_copy(emb_hbm.at[idx], rows.at[b], sem).wait()
            drains[b] = pltpu.async_copy(
                rows.at[b], out_hbm.at[pl.ds(base + c * CH, CH)], out_sems[b])
        for b in range(2):
            if drains[b] is not None:
                drains[b].wait()

    run = pl.kernel(
        body,
        out_type=jax.ShapeDtypeStruct((Tc, D), jnp.float32),
        mesh=plsc.VectorSubcoreMesh(
            core_axis_name="c", subcore_axis_name="s",
            num_cores=_NC, num_subcores=_NS,
        ),
        scratch_types=[
            pltpu.VMEM((PW + _L,), jnp.int32),
            pltpu.VMEM((PW,), jnp.int32),
            pltpu.VMEM((2, CH, D), jnp.float32),
            pltpu.SemaphoreType.DMA,
            pltpu.SemaphoreType.DMA,
            pltpu.SemaphoreType.DMA,
        ],
    )
    return run(x_flat, embed)


def _tc_project_chunk(out_buf, e, wb, scale_arr, row_start, total_rows):
    S, D = e.shape
    MD = wb.shape[0]
    BM = 2048
    nblk = S // BM
    blk0 = row_start // BM
    assert row_start % BM == 0

    def body(*refs):
        s_ref, e_ref, w_ref, o_ref = refs[-4:]
        eb = (e_ref[...] * s_ref[0]).astype(jnp.bfloat16)
        o_ref[...] = lax.dot_general(
            eb, w_ref[...], (((1,), (1,)), ((), ())),
            preferred_element_type=jnp.float32,
        )

    data_specs = [
        pl.BlockSpec(memory_space=pltpu.SMEM),
        pl.BlockSpec((BM, D), lambda i: (i, 0)),
        pl.BlockSpec((MD, D), lambda i: (0, 0)),
    ]
    out_spec = pl.BlockSpec((BM, MD), lambda i, b0=blk0: (b0 + i, 0))
    if out_buf is None:
        return pl.pallas_call(
            body,
            grid=(nblk,),
            in_specs=data_specs,
            out_specs=out_spec,
            out_shape=jax.ShapeDtypeStruct((total_rows, MD), jnp.float32),
        )(scale_arr, e, wb)
    return pl.pallas_call(
        body,
        grid=(nblk,),
        in_specs=[pl.BlockSpec(memory_space=pl.ANY)] + data_specs,
        out_specs=out_spec,
        out_shape=jax.ShapeDtypeStruct((total_rows, MD), jnp.float32),
        input_output_aliases={0: 0},
    )(out_buf, scale_arr, e, wb)


def kernel(x, embed, proj_w, scale):
    B, S = x.shape
    MD = proj_w.shape[0]
    x_flat = x.reshape(-1).astype(jnp.int32)
    wb = proj_w.astype(jnp.bfloat16)
    scale_arr = jnp.asarray(scale, jnp.float32).reshape(1)
    T = B * S
    sizes = [T]
    starts = [0]
    es = [_sc_gather(lax.dynamic_slice_in_dim(x_flat, starts[c], sizes[c]), embed, S)
          for c in range(len(sizes))]
    out = None
    for c in range(len(sizes)):
        out = _tc_project_chunk(out, es[c], wb, scale_arr, starts[c], T)
    return out.reshape(B, S, MD)


# speedup vs baseline: 1.0385x; 1.0019x over previous
import jax
import jax.numpy as jnp
from jax import lax
from jax.experimental import pallas as pl
from jax.experimental.pallas import tpu as pltpu
from jax.experimental.pallas import tpu_sc as plsc

_NC = 2
_NS = 16
_L = 16
_NW = _NC * _NS

_C1 = 36313
_C2 = 27191


def _sc_gather(x_flat, embed, seq_len):
    Tc = x_flat.shape[0]
    V, D = embed.shape
    mod = V - 1
    PW = Tc // _NW
    CH = min(PW, 64)
    NCHUNK = PW // CH
    assert PW % _L == 0 and PW * _NW == Tc and PW % CH == 0
    assert seq_len % PW == 0

    def body(x_hbm, emb_hbm, out_hbm, xbuf, hbuf, rows, sem, sem_o0, sem_o1):
        cid = lax.axis_index("c")
        sid = lax.axis_index("s")
        wid = sid * _NC + cid
        base = wid * PW
        col = lax.rem(base, seq_len)

        pltpu.sync_copy(x_hbm.at[pl.ds(base, PW)], xbuf.at[pl.ds(_L, PW)])

        @pl.when(col != 0)
        def _():
            pltpu.sync_copy(x_hbm.at[pl.ds(base - _L, _L)], xbuf.at[pl.ds(0, _L)])

        pc = (1 << 17) % mod
        lanes = lax.iota(jnp.int32, _L)
        for i in range(PW // _L):
            cur = xbuf[pl.ds(_L + i * _L, _L)]
            prev = xbuf[pl.ds(_L - 1 + i * _L, _L)]
            r = (cur * _C1) ^ (prev * _C2)
            for _ in range(7):
                r = (r >> 17) * pc + (r & 0x1FFFF)
            r = jnp.where(r < 0, r + mod, r)
            r = jnp.where(r < 0, r + mod, r)
            r = jnp.where(r >= mod, r - mod, r)
            r = jnp.where(r >= mod, r - mod, r)
            pos_in_row = col + (i * _L) + lanes
            hbuf[pl.ds(i * _L, _L)] = jnp.where(pos_in_row == 0, mod, r)

        out_sems = (sem_o0, sem_o1)
        drains = [None, None]
        for c in range(NCHUNK):
            b = c % 2
            if drains[b] is not None:
                drains[b].wait()
            idx = hbuf.at[pl.ds(c * CH, CH)]
            pltpu.async_copy(emb_hbm.at[idx], rows.at[b], sem).wait()
            drains[b] = pltpu.async_copy(
                rows.at[b], out_hbm.at[pl.ds(base + c * CH, CH)], out_sems[b])
        for b in range(2):
            if drains[b] is not None:
                drains[b].wait()

    run = pl.kernel(
        body,
        out_type=jax.ShapeDtypeStruct((Tc, D), jnp.float32),
        mesh=plsc.VectorSubcoreMesh(
            core_axis_name="c", subcore_axis_name="s",
            num_cores=_NC, num_subcores=_NS,
        ),
        scratch_types=[
            pltpu.VMEM((PW + _L,), jnp.int32),
            pltpu.VMEM((PW,), jnp.int32),
            pltpu.VMEM((2, CH, D), jnp.float32),
            pltpu.SemaphoreType.DMA,
            pltpu.SemaphoreType.DMA,
            pltpu.SemaphoreType.DMA,
        ],
    )
    return run(x_flat, embed)


def _tc_project_chunk(out_buf, e, wb, scale_arr, row_start, total_rows):
    S, D = e.shape
    MD = wb.shape[0]
    BM = 2048
    nblk = S // BM
    blk0 = row_start // BM
    assert row_start % BM == 0

    def body(*refs):
        s_ref, e_ref, w_ref, o_ref = refs[-4:]
        eb = (e_ref[...] * s_ref[0]).astype(jnp.bfloat16)
        o_ref[...] = lax.dot_general(
            eb, w_ref[...], (((1,), (1,)), ((), ())),
            preferred_element_type=jnp.float32,
        )

    data_specs = [
        pl.BlockSpec(memory_space=pltpu.SMEM),
        pl.BlockSpec((BM, D), lambda i: (i, 0)),
        pl.BlockSpec((MD, D), lambda i: (0, 0)),
    ]
    out_spec = pl.BlockSpec((BM, MD), lambda i, b0=blk0: (b0 + i, 0))
    if out_buf is None:
        return pl.pallas_call(
            body,
            grid=(nblk,),
            in_specs=data_specs,
            out_specs=out_spec,
            out_shape=jax.ShapeDtypeStruct((total_rows, MD), jnp.float32),
        )(scale_arr, e, wb)
    return pl.pallas_call(
        body,
        grid=(nblk,),
        in_specs=[pl.BlockSpec(memory_space=pl.ANY)] + data_specs,
        out_specs=out_spec,
        out_shape=jax.ShapeDtypeStruct((total_rows, MD), jnp.float32),
        input_output_aliases={0: 0},
    )(out_buf, scale_arr, e, wb)


def kernel(x, embed, proj_w, scale):
    B, S = x.shape
    MD = proj_w.shape[0]
    wb = proj_w.astype(jnp.bfloat16)
    scale_arr = jnp.asarray(scale, jnp.float32).reshape(1)
    T = B * S
    e = _sc_gather(x.reshape(-1).astype(jnp.int32), embed, S)
    out = _tc_project_chunk(None, e, wb, scale_arr, 0, T)
    return out.reshape(B, S, MD)


# speedup vs baseline: 1.0401x; 1.0015x over previous
import jax
import jax.numpy as jnp
from jax import lax
from jax.experimental import pallas as pl
from jax.experimental.pallas import tpu as pltpu
from jax.experimental.pallas import tpu_sc as plsc

_NC = 2
_NS = 16
_L = 16
_NW = _NC * _NS

_C1 = 36313
_C2 = 27191


def _sc_gather(x_flat, embed, seq_len):
    Tc = x_flat.shape[0]
    V, D = embed.shape
    mod = V - 1
    PW = Tc // _NW
    CH = min(PW, 64)
    NCHUNK = PW // CH
    assert PW % _L == 0 and PW * _NW == Tc and PW % CH == 0
    assert seq_len % PW == 0

    def body(x_hbm, emb_hbm, out_hbm, xbuf, hbuf, rows, sem, sem_o0, sem_o1):
        cid = lax.axis_index("c")
        sid = lax.axis_index("s")
        wid = sid * _NC + cid
        base = wid * PW
        col = lax.rem(base, seq_len)

        pltpu.sync_copy(x_hbm.at[pl.ds(base, PW)], xbuf.at[pl.ds(_L, PW)])

        @pl.when(col != 0)
        def _():
            pltpu.sync_copy(x_hbm.at[pl.ds(base - _L, _L)], xbuf.at[pl.ds(0, _L)])

        pc = (1 << 17) % mod
        lanes = lax.iota(jnp.int32, _L)
        for i in range(PW // _L):
            cur = xbuf[pl.ds(_L + i * _L, _L)]
            prev = xbuf[pl.ds(_L - 1 + i * _L, _L)]
            r = (cur * _C1) ^ (prev * _C2)
            for _ in range(7):
                r = (r >> 17) * pc + (r & 0x1FFFF)
            r = jnp.where(r < 0, r + mod, r)
            r = jnp.where(r < 0, r + mod, r)
            r = jnp.where(r >= mod, r - mod, r)
            r = jnp.where(r >= mod, r - mod, r)
            pos_in_row = col + (i * _L) + lanes
            hbuf[pl.ds(i * _L, _L)] = jnp.where(pos_in_row == 0, mod, r)

        out_sems = (sem_o0, sem_o1)
        drains = [None, None]
        for c in range(NCHUNK):
            b = c % 2
            if drains[b] is not None:
                drains[b].wait()
            idx = hbuf.at[pl.ds(c * CH, CH)]
            pltpu.async_copy(emb_hbm.at[idx], rows.at[b], sem).wait()
            drains[b] = pltpu.async_copy(
                rows.at[b], out_hbm.at[pl.ds(base + c * CH, CH)], out_sems[b])
        for b in range(2):
            if drains[b] is not None:
                drains[b].wait()

    run = pl.kernel(
        body,
        out_type=jax.ShapeDtypeStruct((Tc, D), jnp.float32),
        mesh=plsc.VectorSubcoreMesh(
            core_axis_name="c", subcore_axis_name="s",
            num_cores=_NC, num_subcores=_NS,
        ),
        scratch_types=[
            pltpu.VMEM((PW + _L,), jnp.int32),
            pltpu.VMEM((PW,), jnp.int32),
            pltpu.VMEM((2, CH, D), jnp.float32),
            pltpu.SemaphoreType.DMA,
            pltpu.SemaphoreType.DMA,
            pltpu.SemaphoreType.DMA,
        ],
    )
    return run(x_flat, embed)


def _tc_project(e, wb, scale_arr):
    T, D = e.shape
    MD = wb.shape[0]
    BM = 2048

    def body(s_ref, e_ref, w_ref, o_ref):
        eb = (e_ref[...] * s_ref[0]).astype(jnp.bfloat16)
        o_ref[...] = lax.dot_general(
            eb, w_ref[...], (((1,), (1,)), ((), ())),
            preferred_element_type=jnp.float32,
        )

    return pl.pallas_call(
        body,
        grid=(T // BM,),
        in_specs=[
            pl.BlockSpec(memory_space=pltpu.SMEM),
            pl.BlockSpec((BM, D), lambda i: (i, 0)),
            pl.BlockSpec((MD, D), lambda i: (0, 0)),
        ],
        out_specs=pl.BlockSpec((BM, MD), lambda i: (i, 0)),
        out_shape=jax.ShapeDtypeStruct((T, MD), jnp.float32),
    )(scale_arr, e, wb)


def kernel(x, embed, proj_w, scale):
    B, S = x.shape
    MD = proj_w.shape[0]
    wb = proj_w.astype(jnp.bfloat16)
    scale_arr = jnp.asarray(scale, jnp.float32).reshape(1)
    e = _sc_gather(x.reshape(-1).astype(jnp.int32), embed, S)
    out = _tc_project(e, wb, scale_arr)
    return out.reshape(B, S, MD)
